# baseline (speedup 1.0000x reference)
import functools

import jax
import jax.numpy as jnp
from jax import lax
from jax.experimental import pallas as pl
from jax.experimental.pallas import tpu as pltpu
from jax.experimental.pallas import tpu_sc as plsc

N = 10000
N_PAD = 10240
E = 320000
D = 128
NC, NS = 2, 16
NW = NC * NS
B = 80
NBT = 126
E_PAD = NW * NBT * B
ROW_BLK = 128
NBLK = 10


def _proj_body(x_ref, wg_ref, att_ref, h_ref, a2_ref):
    h = jnp.dot(x_ref[...], wg_ref[...], preferred_element_type=jnp.float32)
    h_ref[...] = h
    a2_ref[...] = jnp.dot(h, att_ref[...], preferred_element_type=jnp.float32)


_proj_call = pl.pallas_call(
    _proj_body,
    grid=(NBLK,),
    in_specs=[
        pl.BlockSpec((N // NBLK, D), lambda i: (i, 0)),
        pl.BlockSpec((D, D), lambda i: (0, 0)),
        pl.BlockSpec((D, 8), lambda i: (0, 0)),
    ],
    out_specs=[
        pl.BlockSpec((N // NBLK, D), lambda i: (i, 0)),
        pl.BlockSpec((N // NBLK, 8), lambda i: (i, 0)),
    ],
    out_shape=[
        jax.ShapeDtypeStruct((N, D), jnp.float32),
        jax.ShapeDtypeStruct((N, 8), jnp.float32),
    ],
)


_mesh = plsc.VectorSubcoreMesh(core_axis_name="c", subcore_axis_name="s")


@functools.partial(
    pl.kernel,
    mesh=_mesh,
    compiler_params=pltpu.CompilerParams(needs_layout_passes=False),
    out_type=(jax.ShapeDtypeStruct((NC, N_PAD, D), jnp.float32),
              jax.ShapeDtypeStruct((NW, N_PAD), jnp.float32)),
    scratch_types=[
        pltpu.VMEM((B,), jnp.int32),
        pltpu.VMEM((B,), jnp.int32),
        pltpu.VMEM((B,), jnp.int32),
        pltpu.VMEM((B,), jnp.int32),
        pltpu.VMEM((B,), jnp.int32),
        pltpu.VMEM((B,), jnp.int32),
        pltpu.VMEM((B,), jnp.float32),
        pltpu.VMEM((B,), jnp.float32),
        pltpu.VMEM((B,), jnp.float32),
        pltpu.VMEM((B,), jnp.float32),
        pltpu.VMEM((B,), jnp.float32),
        pltpu.VMEM((B,), jnp.float32),
        pltpu.VMEM((B,), jnp.int32),
        pltpu.VMEM((B,), jnp.int32),
        pltpu.VMEM((B,), jnp.int32),
        pltpu.VMEM((B,), jnp.float32),
        pltpu.VMEM((B, D), jnp.float32),
        pltpu.VMEM((B, D), jnp.float32),
        pltpu.VMEM((B, D), jnp.float32),
        pltpu.VMEM((N_PAD,), jnp.float32),
        pltpu.VMEM((16,), jnp.int32),
        pltpu.VMEM((16,), jnp.float32),
        pltpu.VMEM_SHARED((N_PAD, D), jnp.float32),
        pltpu.VMEM_SHARED((N,), jnp.float32),
        pltpu.VMEM_SHARED((N,), jnp.float32),
        pltpu.SemaphoreType.DMA,
        pltpu.SemaphoreType.DMA,
        pltpu.SemaphoreType.DMA,
        pltpu.SemaphoreType.DMA,
        pltpu.SemaphoreType.DMA,
        pltpu.SemaphoreType.DMA,
        pltpu.SemaphoreType.DMA,
        pltpu.SemaphoreType.DMA,
        pltpu.SemaphoreType.DMA,
        pltpu.SemaphoreType.DMA,
        pltpu.SemaphoreType.DMA,
        pltpu.SemaphoreType.DMA,
        pltpu.SemaphoreType.DMA,
        pltpu.SemaphoreType.DMA,
        pltpu.SemaphoreType.DMA,
    ],
)
def _edge_kernel(h_hbm, asrc_hbm, adst_hbm, src_hbm, dst_hbm,
                 out_hbm, den_hbm,
                 si0, si1, si2, di0, di1, di2, av0, av1, av2,
                 dv0, dv1, dv2, dc0, dc1, dc2, wv, gv0, gv1, gv2,
                 dnm, bi, bf,
                 acc, aSsh, aDsh,
                 ia0, ia1, ia2, ib0, ib1, ib2, aa0, aa1, aa2,
                 gg0, gg1, gg2, ss0, ss1, ss2):
    c = lax.axis_index("c")
    s = lax.axis_index("s")
    si = (si0, si1, si2)
    di = (di0, di1, di2)
    av = (av0, av1, av2)
    dv = (dv0, dv1, dv2)
    dsc = (dc0, dc1, dc2)
    gv = (gv0, gv1, gv2)
    isa = (ia0, ia1, ia2)
    isb = (ib0, ib1, ib2)
    asem = (aa0, aa1, aa2)
    gsem = (gg0, gg1, gg2)
    ssem = (ss0, ss1, ss2)

    def zrow(j, carry):
        zeros16 = jnp.zeros((16,), jnp.float32)
        for k in range(D // 16):
            gv0[j, pl.ds(k * 16, 16)] = zeros16
        return carry

    lax.fori_loop(0, B, zrow, 0)

    def zden(j, carry):
        dnm[pl.ds(j * 16, 16)] = jnp.zeros((16,), jnp.float32)
        return carry

    lax.fori_loop(0, N_PAD // 16, zden, 0)
    for i in range(640 // B):
        pltpu.sync_copy(gv0.at[pl.ds(0, B)],
                        acc.at[pl.ds(s * 640 + i * B, B)])
    @pl.when(s == 0)
    def _():
        pltpu.sync_copy(asrc_hbm, aSsh)
        pltpu.sync_copy(adst_hbm, aDsh)

    plsc.subcore_barrier()

    wid = c * NS + s
    start = wid * NBT

    def fetch_src(jb, r):
        pltpu.async_copy(src_hbm.at[pl.ds(jb * B, B)], si[r], isa[r])

    def fetch_dst(jb, r):
        pltpu.async_copy(dst_hbm.at[pl.ds(jb * B, B)], di[r], isb[r])

    def launch(jb, r):
        pltpu.make_async_copy(src_hbm.at[pl.ds(jb * B, B)], si[r],
                              isa[r]).wait()
        pltpu.make_async_copy(dst_hbm.at[pl.ds(jb * B, B)], di[r],
                              isb[r]).wait()
        pltpu.async_copy(h_hbm.at[si[r]], gv[r], gsem[r])
        pltpu.async_copy(aSsh.at[si[r]], av[r], asem[r])
        pltpu.async_copy(aDsh.at[di[r]], dv[r], asem[r])

    def wait_gather(r):
        pltpu.make_async_copy(h_hbm.at[si[r]], gv[r], gsem[r]).wait()

    def scatter(r):
        pltpu.async_copy(gv[r], acc.at[dsc[r]], ssem[r], add=True)

    def wait_scatter(r):
        pltpu.make_async_copy(gv[r], acc.at[dsc[r]], ssem[r]).wait()

    def w_denom(r):
        pltpu.make_async_copy(aSsh.at[si[r]], av[r], asem[r]).wait()
        pltpu.make_async_copy(aDsh.at[di[r]], dv[r], asem[r]).wait()
        for k in range(B // 16):
            dvec = di[r][pl.ds(k * 16, 16)]
            a = av[r][pl.ds(k * 16, 16)] + dv[r][pl.ds(k * 16, 16)]
            e = jnp.where(a >= 0.0, a, 0.2 * a)
            w = jnp.exp(e)
            wv[pl.ds(k * 16, 16)] = w
            iota = lax.iota(jnp.int32, 16)
            d_s, w_s = plsc.sort_key_val(dvec, w)
            bi[...] = d_s
            bf[...] = plsc.cumsum(w_s)
            prev = plsc.load_gather(bi, [jnp.maximum(iota - 1, 0)])
            nxt = plsc.load_gather(bi, [jnp.minimum(iota + 1, 15)])
            first = (iota == 0) | (d_s != prev)
            last = (iota == 15) | (d_s != nxt)
            segstart = plsc.cummax(jnp.where(first, iota, 0))
            csum = bf[...]
            sprev = plsc.load_gather(bf, [jnp.maximum(segstart - 1, 0)])
            total = csum - jnp.where(segstart == 0, 0.0, sprev)
            plsc.addupdate_scatter(dnm, [d_s], total, mask=last)

    def scale(r):
        gvp = gv[r]
        for k in range(B // 16):
            dsc[r][pl.ds(k * 16, 16)] = di[r][pl.ds(k * 16, 16)]

        @plsc.parallel_loop(0, B, unroll=8)
        def _(j):
            wj = plsc.load_gather(wv, [jnp.full((16,), j, jnp.int32)])
            for k in range(D // 16):
                gvp[j, pl.ds(k * 16, 16)] = gvp[j, pl.ds(k * 16, 16)] * wj

    def step(jb, r, pre):
        rn = (r + 1) % 3
        rn2 = (r + 2) % 3
        if pre in ("first", "mid"):
            fetch_src(jb + 2, rn2)
            fetch_dst(jb + 2, rn2)
        if pre != "last":
            launch(jb + 1, rn)
        w_denom(r)
        wait_gather(r)
        scale(r)
        scatter(r)
        if pre != "first":
            wait_scatter(rn2)
    fetch_src(start, 0)
    fetch_dst(start, 0)
    fetch_src(start + 1, 1)
    fetch_dst(start + 1, 1)
    launch(start, 0)
    step(start, 0, "first")
    def triple_body(i, carry):
        jb = start + 1 + 3 * i
        step(jb, 1, "mid")
        step(jb + 1, 2, "mid")
        step(jb + 2, 0, "mid")
        return carry

    lax.fori_loop(0, (NBT - 3) // 3, triple_body, 0)
    step(start + NBT - 2, 1, "tail")
    step(start + NBT - 1, 2, "last")
    wait_scatter(2)
    plsc.subcore_barrier()
    for i in range(5):
        r0 = s * 640 + i * ROW_BLK
        pltpu.sync_copy(acc.at[pl.ds(r0, ROW_BLK)],
                        out_hbm.at[c, pl.ds(r0, ROW_BLK)])
    pltpu.sync_copy(dnm, den_hbm.at[wid])


def _final_body(p_ref, d_ref, bias_ref, wfc_ref, bfc_ref, o_ref):
    p = p_ref[...]
    feats = p[0] + p[1]
    den = jnp.sum(d_ref[...], axis=1, keepdims=True) + 1e-16
    g = jnp.maximum(feats / den + bias_ref[...], 0.0)
    o_ref[...] = (jnp.dot(g, wfc_ref[...], preferred_element_type=jnp.float32)
                  + bfc_ref[...])


_final_call = pl.pallas_call(
    _final_body,
    grid=(NBLK,),
    in_specs=[
        pl.BlockSpec((NC, N // NBLK, D), lambda i: (0, i, 0)),
        pl.BlockSpec((N // NBLK, NW), lambda i: (i, 0)),
        pl.BlockSpec((1, D), lambda i: (0, 0)),
        pl.BlockSpec((D, D), lambda i: (0, 0)),
        pl.BlockSpec((1, D), lambda i: (0, 0)),
    ],
    out_specs=pl.BlockSpec((N // NBLK, D), lambda i: (i, 0)),
    out_shape=jax.ShapeDtypeStruct((N, D), jnp.float32),
)


def kernel(x, edge_index, W_gat, att_src, att_dst, bias_gat, W_fc, b_fc):
    att_pack = jnp.zeros((D, 8), jnp.float32)
    att_pack = att_pack.at[:, 0].set(att_src[0]).at[:, 1].set(att_dst[0])
    h, a2 = _proj_call(x, W_gat, att_pack)
    a_src = a2[:, 0]
    a_dst = a2[:, 1]
    pidx = jnp.arange(E_PAD - E, dtype=jnp.int32)
    src2 = jnp.concatenate([edge_index[0], pidx % N])
    dst2 = jnp.concatenate([edge_index[1], N + pidx % (N_PAD - N)])
    partials, dens = _edge_kernel(h, a_src, a_dst, src2, dst2)
    return _final_call(partials, dens.T, bias_gat.reshape(1, D), W_fc,
                       b_fc.reshape(1, D))
